# baseline (speedup 1.0000x reference)
import functools

import jax
import jax.numpy as jnp
import numpy as np
from jax import lax
from jax.experimental import pallas as pl
from jax.experimental.pallas import tpu as pltpu
from jax.experimental.pallas import tpu_sc as plsc

_L = 16
_CB = 32
_GI = 104


@jax.jit
def kernel(x_num, x_cat, offsets, table):
    B, NN = x_num.shape
    F = x_cat.shape[1]
    V, D = table.shape
    OUTW = NN + F * D
    EW = F * D
    MR = EW // _L
    SROWS = _CB // 8
    SW = OUTW * 8

    info = plsc.get_sparse_core_info()
    NC, NS = info.num_cores, info.num_subcores
    NW = NC * NS
    assert B % (NW * _CB * 2) == 0
    rows_per_w = B // NW
    n_chunks = rows_per_w // _CB
    NIDX = _CB * F
    NG = NIDX // _GI

    off_tiled = jnp.tile(offsets, _CB)
    p = np.arange(_CB * NN)
    t_srow = jnp.asarray(p // NN // 8, jnp.int32)
    t_scol = jnp.asarray(OUTW * (p // NN % 8) + p % NN, jnp.int32)

    mesh = plsc.VectorSubcoreMesh(core_axis_name="c", subcore_axis_name="s")

    scratch = [
        pltpu.VMEM((NIDX,), jnp.int32),
        pltpu.VMEM((NIDX,), jnp.int32),
        pltpu.VMEM((NIDX,), jnp.int32),
        pltpu.VMEM((NIDX,), jnp.int32),
        pltpu.VMEM((NIDX, D), jnp.float32),
        pltpu.VMEM((NIDX, D), jnp.float32),
        pltpu.VMEM((_CB * NN,), jnp.float32),
        pltpu.VMEM((SROWS, SW), jnp.float32),
        pltpu.VMEM((SROWS, SW), jnp.float32),
        pltpu.VMEM((_CB * NN,), jnp.int32),
        pltpu.VMEM((_CB * NN,), jnp.int32),
    ] + [pltpu.SemaphoreType.DMA] * (2 * NG + 2)

    @functools.partial(
        pl.kernel,
        out_type=jax.ShapeDtypeStruct((B // 8, SW), jnp.float32),
        mesh=mesh,
        scratch_types=scratch,
        compiler_params=pltpu.CompilerParams(
            use_tc_tiling_on_sc=False, needs_layout_passes=False),
    )
    def run(xnum_hbm, xcat_hbm, off_hbm, table_hbm, srow_hbm, scol_hbm,
            out_hbm,
            xcat_v, off_v, idx_a, idx_b, emb_a, emb_b, xnum_v,
            stag_a, stag_b, srow_v, scol_v, *sems):
        sga = sems[:NG]
        sgb = sems[NG:2 * NG]
        sem_wa = sems[2 * NG]
        sem_wb = sems[2 * NG + 1]
        cid = lax.axis_index("c")
        sid = lax.axis_index("s")
        wid = sid * NC + cid

        pltpu.sync_copy(off_hbm, off_v)
        pltpu.sync_copy(srow_hbm, srow_v)
        pltpu.sync_copy(scol_hbm, scol_v)

        def gather_cp(idx_v, emb_v, sg, g):
            return pltpu.make_async_copy(
                table_hbm.at[idx_v.at[pl.ds(g * _GI, _GI)]],
                emb_v.at[pl.ds(g * _GI, _GI)], sg[g])

        def write_cp(staging, sem_w, c):
            crow = wid * (rows_per_w // 8) + c * SROWS
            return pltpu.make_async_copy(
                staging, out_hbm.at[pl.ds(crow, SROWS)], sem_w)

        def stage(idx_v, emb_v, sg, c):
            pltpu.sync_copy(xcat_hbm.at[wid * n_chunks + c], xcat_v)

            def add_body(k, carry):
                s = pl.ds(k * _L, _L)
                idx_v[s] = xcat_v[s] + off_v[s]
                return carry
            lax.fori_loop(0, NIDX // _L, add_body, 0)
            for g in range(NG):
                gather_cp(idx_v, emb_v, sg, g).start()

        def consume(idx_v, emb_v, sg, staging, sem_w, c, first):
            for g in range(NG):
                gather_cp(idx_v, emb_v, sg, g).wait()
            pltpu.sync_copy(xnum_hbm.at[wid * n_chunks + c], xnum_v)

            @pl.when(jnp.logical_not(first))
            def _():
                write_cp(staging, sem_w, c - 2).wait()

            def num_body(k, carry):
                s = pl.ds(k * _L, _L)
                plsc.store_scatter(staging, [srow_v[s], scol_v[s]], xnum_v[s])
                return carry
            lax.fori_loop(0, _CB * NN // _L, num_body, 0)

            def row_body(b, carry):
                r = b // 8
                col0 = OUTW * (b % 8) + NN
                for j in range(MR):
                    v = emb_v[F * b + j // 2, pl.ds(_L * (j % 2), _L)]
                    staging[r, pl.ds(col0 + _L * j, _L)] = v
                return carry
            lax.fori_loop(0, _CB, row_body, 0)
            write_cp(staging, sem_w, c).start()

        A = (idx_a, emb_a, sga)
        B_ = (idx_b, emb_b, sgb)

        stage(*A, 0)

        def pair_body(t, carry):
            c0 = 2 * t
            stage(*B_, c0 + 1)
            consume(*A, stag_a, sem_wa, c0, t == 0)

            @pl.when(c0 + 2 < n_chunks)
            def _():
                stage(*A, c0 + 2)
            consume(*B_, stag_b, sem_wb, c0 + 1, t == 0)
            return carry

        lax.fori_loop(0, n_chunks // 2, pair_body, 0)
        write_cp(stag_a, sem_wa, n_chunks - 2).wait()
        write_cp(stag_b, sem_wb, n_chunks - 1).wait()

    out = run(x_num.reshape(B // _CB, _CB * NN),
              x_cat.reshape(B // _CB, NIDX),
              off_tiled, table, t_srow, t_scol)
    return out.reshape(B, OUTW)
